# baseline (speedup 1.0000x reference)
import jax
import jax.numpy as jnp
from jax.experimental import pallas as pl
from jax.experimental.pallas import tpu as pltpu

S = 256
W = 32

_HIGH = jax.lax.Precision.HIGHEST


def _split_bf16(v):
    hi = v.astype(jnp.bfloat16)
    lo = (v - hi.astype(jnp.float32)).astype(jnp.bfloat16)
    return hi, lo


def _body(bases_ref, lasts_ref, x_ref, batch_ref, W1_ref, b1_ref, W2r_ref,
          out_ref, m_ref, d_ref):
    i = pl.program_id(0)
    nb = pl.num_programs(0)

    @pl.when(i == 0)
    def _init():
        out_ref[...] = jnp.zeros_like(out_ref)
        m_ref[...] = jnp.full(m_ref.shape, -jnp.inf, jnp.float32)
        d_ref[...] = jnp.zeros_like(d_ref)

    x = x_ref[...]
    R = x.shape[0]
    xh, xl = _split_bf16(x)
    W1h = W1_ref[...].astype(jnp.bfloat16)
    pre = jax.lax.dot_general(
        xh, W1h, (((1,), (0,)), ((), ())),
        preferred_element_type=jnp.float32,
    ) + jax.lax.dot_general(
        xl, W1h, (((1,), (0,)), ((), ())),
        preferred_element_type=jnp.float32,
    )
    h = jnp.tanh(pre + b1_ref[...])
    s = jax.lax.dot_general(
        h, W2r_ref[...], (((1,), (1,)), ((), ())), precision=_HIGH
    )

    ids = batch_ref[...].reshape(R, 1)
    neg = jnp.float32(-jnp.inf)
    base = (bases_ref[i] // 8) * 8
    span_ok = (lasts_ref[i] - base) < W

    def seg_update(mask, row_valid, w, start):
        m_blk = jnp.max(jnp.where(mask, s, neg), axis=0, keepdims=True)
        m_old = m_ref[pl.ds(start, w), :].reshape(1, w)
        m_new = jnp.maximum(m_old, m_blk)
        scale = jnp.where(m_new == neg, 0.0, jnp.exp(m_old - m_new))
        m_ref[pl.ds(start, w), :] = m_new.reshape(w, 1)
        m_row = jnp.max(jnp.where(mask, m_new, neg), axis=1, keepdims=True)
        e = jnp.where(row_valid, jnp.exp(s - m_row), 0.0)
        d_blk = jnp.sum(jnp.where(mask, e, 0.0), axis=0, keepdims=True)
        scale_c = scale.reshape(w, 1)
        d_ref[pl.ds(start, w), :] = (
            d_ref[pl.ds(start, w), :] * scale_c + d_blk.reshape(w, 1)
        )
        mh = mask.astype(jnp.bfloat16)
        xh, xl = _split_bf16(x * e)
        p = jax.lax.dot_general(
            mh, xh, (((0,), (0,)), ((), ())),
            preferred_element_type=jnp.float32,
        ) + jax.lax.dot_general(
            mh, xl, (((0,), (0,)), ((), ())),
            preferred_element_type=jnp.float32,
        )
        out_ref[pl.ds(start, w), :] = (
            out_ref[pl.ds(start, w), :] * scale_c + p
        )

    @pl.when(span_ok)
    def _fast():
        rel = ids - base
        seg = jax.lax.broadcasted_iota(jnp.int32, (R, W), 1)
        mask = rel == seg
        row_valid = (rel >= 0) & (rel < W)
        seg_update(mask, row_valid, W, base)

    @pl.when(jnp.logical_not(span_ok))
    def _slow():
        seg = jax.lax.broadcasted_iota(jnp.int32, (R, S), 1)
        mask = ids == seg
        row_valid = ids < S
        seg_update(mask, row_valid, S, 0)

    @pl.when(i == nb - 1)
    def _finish():
        out_ref[...] = out_ref[...] / (d_ref[...] + 1e-16)


def _pick_block(n: int) -> int:
    for r in range(5000, 7, -8):
        if n % r == 0:
            return r
    return 0


def kernel(x, W1, b1, W2, batch):
    N, D = x.shape
    R = _pick_block(N)
    batch = batch.astype(jnp.int32)
    if R == 0:
        R = 2048
        pad = (-N) % R
        x = jnp.pad(x, ((0, pad), (0, 0)))
        batch = jnp.concatenate(
            [batch, jnp.full((pad,), S, dtype=jnp.int32)]
        )
        N = N + pad
    NB = N // R

    b2 = batch.reshape(NB, R)
    bases = b2[:, 0]
    lasts = b2[:, -1]
    batch3 = b2.reshape(NB, R, 1)
    b1r = b1.reshape(1, D)
    W2r = W2.reshape(1, D)

    out = pl.pallas_call(
        _body,
        grid_spec=pltpu.PrefetchScalarGridSpec(
            num_scalar_prefetch=2,
            grid=(NB,),
            in_specs=[
                pl.BlockSpec((R, D), lambda i, *_: (i, 0)),
                pl.BlockSpec((1, R, 1), lambda i, *_: (i, 0, 0)),
                pl.BlockSpec((D, D), lambda i, *_: (0, 0)),
                pl.BlockSpec((1, D), lambda i, *_: (0, 0)),
                pl.BlockSpec((1, D), lambda i, *_: (0, 0)),
            ],
            out_specs=pl.BlockSpec((S, D), lambda i, *_: (0, 0)),
            scratch_shapes=[
                pltpu.VMEM((S, 1), jnp.float32),
                pltpu.VMEM((S, 1), jnp.float32),
            ],
        ),
        out_shape=jax.ShapeDtypeStruct((S, D), jnp.float32),
    )(bases, lasts, x, batch3, W1, b1r, W2r)
    return out


# speedup vs baseline: 1.2239x; 1.2239x over previous
import jax
import jax.numpy as jnp
from jax.experimental import pallas as pl
from jax.experimental.pallas import tpu as pltpu

S = 256
W = 32

_HIGH = jax.lax.Precision.HIGHEST


def _split_bf16(v):
    hi = v.astype(jnp.bfloat16)
    lo = (v - hi.astype(jnp.float32)).astype(jnp.bfloat16)
    return hi, lo


def _body(bases_ref, lasts_ref, x_ref, batch_ref, W1_ref, b1_ref, W2r_ref,
          out_ref, m_ref, d_ref):
    i = pl.program_id(0)
    nb = pl.num_programs(0)

    @pl.when(i == 0)
    def _init():
        out_ref[...] = jnp.zeros_like(out_ref)
        m_ref[...] = jnp.full(m_ref.shape, -jnp.inf, jnp.float32)
        d_ref[...] = jnp.zeros_like(d_ref)

    R = x_ref.shape[0]
    xh, xl = _split_bf16(x_ref[...])
    W1h = W1_ref[...].astype(jnp.bfloat16)
    pre = jax.lax.dot_general(
        xh, W1h, (((1,), (0,)), ((), ())),
        preferred_element_type=jnp.float32,
    ) + jax.lax.dot_general(
        xl, W1h, (((1,), (0,)), ((), ())),
        preferred_element_type=jnp.float32,
    )
    h = jnp.tanh((pre + b1_ref[...]).astype(jnp.bfloat16))
    s = jax.lax.dot_general(
        h.astype(jnp.float32), W2r_ref[...], (((1,), (1,)), ((), ())),
        precision=_HIGH,
    )

    ids = batch_ref[...].reshape(R, 1)
    neg = jnp.float32(-jnp.inf)
    base = (bases_ref[i] // 8) * 8
    span_ok = (lasts_ref[i] - base) < W

    def seg_update(mask, row_valid, w, start):
        m_blk = jnp.max(jnp.where(mask, s, neg), axis=0, keepdims=True)
        m_old = m_ref[pl.ds(start, w), :].reshape(1, w)
        m_new = jnp.maximum(m_old, m_blk)
        scale = jnp.where(m_new == neg, 0.0, jnp.exp(m_old - m_new))
        m_ref[pl.ds(start, w), :] = m_new.reshape(w, 1)
        m_row = jnp.max(jnp.where(mask, m_new, neg), axis=1, keepdims=True)
        ex = jnp.exp(s - m_row)
        e = ex if row_valid is None else jnp.where(row_valid, ex, 0.0)
        d_blk = jnp.sum(jnp.where(mask, e, 0.0), axis=0, keepdims=True)
        scale_c = scale.reshape(w, 1)
        d_ref[pl.ds(start, w), :] = (
            d_ref[pl.ds(start, w), :] * scale_c + d_blk.reshape(w, 1)
        )
        mh = mask.astype(jnp.bfloat16)
        xh, xl = _split_bf16(x_ref[...] * e)
        p = jax.lax.dot_general(
            mh, xh, (((0,), (0,)), ((), ())),
            preferred_element_type=jnp.float32,
        ) + jax.lax.dot_general(
            mh, xl, (((0,), (0,)), ((), ())),
            preferred_element_type=jnp.float32,
        )
        out_ref[pl.ds(start, w), :] = (
            out_ref[pl.ds(start, w), :] * scale_c + p
        )

    @pl.when(span_ok)
    def _fast():
        rel = ids - base
        seg = jax.lax.broadcasted_iota(jnp.int32, (R, W), 1)
        mask = rel == seg
        seg_update(mask, None, W, base)

    @pl.when(jnp.logical_not(span_ok))
    def _slow():
        seg = jax.lax.broadcasted_iota(jnp.int32, (R, S), 1)
        mask = ids == seg
        row_valid = ids < S
        seg_update(mask, row_valid, S, 0)

    @pl.when(i == nb - 1)
    def _finish():
        out_ref[...] = out_ref[...] / (d_ref[...] + 1e-16)


def _pick_block(n: int) -> int:
    for r in range(4000, 7, -8):
        if n % r == 0:
            return r
    return 0


def kernel(x, W1, b1, W2, batch):
    N, D = x.shape
    R = _pick_block(N)
    batch = batch.astype(jnp.int32)
    if R == 0:
        R = 2048
        pad = (-N) % R
        x = jnp.pad(x, ((0, pad), (0, 0)))
        batch = jnp.concatenate(
            [batch, jnp.full((pad,), S, dtype=jnp.int32)]
        )
        N = N + pad
    NB = N // R

    b2 = batch.reshape(NB, R)
    bases = b2[:, 0]
    lasts = b2[:, -1]
    batch3 = b2.reshape(NB, R, 1)
    b1r = b1.reshape(1, D)
    W2r = W2.reshape(1, D)

    out = pl.pallas_call(
        _body,
        grid_spec=pltpu.PrefetchScalarGridSpec(
            num_scalar_prefetch=2,
            grid=(NB,),
            in_specs=[
                pl.BlockSpec((R, D), lambda i, *_: (i, 0)),
                pl.BlockSpec((1, R, 1), lambda i, *_: (i, 0, 0)),
                pl.BlockSpec((D, D), lambda i, *_: (0, 0)),
                pl.BlockSpec((1, D), lambda i, *_: (0, 0)),
                pl.BlockSpec((1, D), lambda i, *_: (0, 0)),
            ],
            out_specs=pl.BlockSpec((S, D), lambda i, *_: (0, 0)),
            scratch_shapes=[
                pltpu.VMEM((S, 1), jnp.float32),
                pltpu.VMEM((S, 1), jnp.float32),
            ],
        ),
        out_shape=jax.ShapeDtypeStruct((S, D), jnp.float32),
    )(bases, lasts, x, batch3, W1, b1r, W2r)
    return out


# speedup vs baseline: 1.2511x; 1.0222x over previous
import jax
import jax.numpy as jnp
from jax.experimental import pallas as pl
from jax.experimental.pallas import tpu as pltpu

S = 256
W = 32

_HIGH = jax.lax.Precision.HIGHEST


def _split_bf16(v):
    hi = v.astype(jnp.bfloat16)
    lo = (v - hi.astype(jnp.float32)).astype(jnp.bfloat16)
    return hi, lo


def _body(bases_ref, lasts_ref, x_ref, batch_ref, W1_ref, b1_ref, W2r_ref,
          out_ref, m_ref, d_ref):
    i = pl.program_id(0)
    nb = pl.num_programs(0)

    @pl.when(i == 0)
    def _init():
        out_ref[...] = jnp.zeros_like(out_ref)
        m_ref[...] = jnp.full(m_ref.shape, -jnp.inf, jnp.float32)
        d_ref[...] = jnp.zeros_like(d_ref)

    R = x_ref.shape[0]
    xh, xl = _split_bf16(x_ref[...])
    W1h = W1_ref[...].astype(jnp.bfloat16)
    pre = jax.lax.dot_general(
        xh, W1h, (((1,), (0,)), ((), ())),
        preferred_element_type=jnp.float32,
    ) + jax.lax.dot_general(
        xl, W1h, (((1,), (0,)), ((), ())),
        preferred_element_type=jnp.float32,
    )
    h = jnp.tanh((pre + b1_ref[...]).astype(jnp.bfloat16))
    s = jax.lax.dot_general(
        h.astype(jnp.float32), W2r_ref[...], (((1,), (1,)), ((), ())),
        precision=_HIGH,
    )

    ids = batch_ref[...].reshape(R, 1)
    neg = jnp.float32(-jnp.inf)
    base = (bases_ref[i] // 8) * 8
    span_ok = (lasts_ref[i] - base) < W

    def seg_update(mask, row_valid, w, start):
        m_blk = jnp.max(jnp.where(mask, s, neg), axis=0, keepdims=True)
        m_old = m_ref[pl.ds(start, w), :].reshape(1, w)
        m_new = jnp.maximum(m_old, m_blk)
        scale = jnp.where(m_new == neg, 0.0, jnp.exp(m_old - m_new))
        m_ref[pl.ds(start, w), :] = m_new.reshape(w, 1)
        m_row = jnp.max(jnp.where(mask, m_new, neg), axis=1, keepdims=True)
        ex = jnp.exp(s - m_row)
        e = ex if row_valid is None else jnp.where(row_valid, ex, 0.0)
        d_blk = jnp.sum(jnp.where(mask, e, 0.0), axis=0, keepdims=True)
        scale_c = scale.reshape(w, 1)
        d_ref[pl.ds(start, w), :] = (
            d_ref[pl.ds(start, w), :] * scale_c + d_blk.reshape(w, 1)
        )
        mh = mask.astype(jnp.bfloat16)
        xe = (x_ref[...] * e).astype(jnp.bfloat16)
        p = jax.lax.dot_general(
            mh, xe, (((0,), (0,)), ((), ())),
            preferred_element_type=jnp.float32,
        )
        out_ref[pl.ds(start, w), :] = (
            out_ref[pl.ds(start, w), :] * scale_c + p
        )

    @pl.when(span_ok)
    def _fast():
        rel = ids - base
        seg = jax.lax.broadcasted_iota(jnp.int32, (R, W), 1)
        mask = rel == seg
        seg_update(mask, None, W, base)

    @pl.when(jnp.logical_not(span_ok))
    def _slow():
        seg = jax.lax.broadcasted_iota(jnp.int32, (R, S), 1)
        mask = ids == seg
        row_valid = ids < S
        seg_update(mask, row_valid, S, 0)

    @pl.when(i == nb - 1)
    def _finish():
        out_ref[...] = out_ref[...] / (d_ref[...] + 1e-16)


def _pick_block(n: int) -> int:
    for r in range(4000, 7, -8):
        if n % r == 0:
            return r
    return 0


def kernel(x, W1, b1, W2, batch):
    N, D = x.shape
    R = _pick_block(N)
    batch = batch.astype(jnp.int32)
    if R == 0:
        R = 2048
        pad = (-N) % R
        x = jnp.pad(x, ((0, pad), (0, 0)))
        batch = jnp.concatenate(
            [batch, jnp.full((pad,), S, dtype=jnp.int32)]
        )
        N = N + pad
    NB = N // R

    b2 = batch.reshape(NB, R)
    bases = b2[:, 0]
    lasts = b2[:, -1]
    batch3 = b2.reshape(NB, R, 1)
    b1r = b1.reshape(1, D)
    W2r = W2.reshape(1, D)

    out = pl.pallas_call(
        _body,
        grid_spec=pltpu.PrefetchScalarGridSpec(
            num_scalar_prefetch=2,
            grid=(NB,),
            in_specs=[
                pl.BlockSpec((R, D), lambda i, *_: (i, 0)),
                pl.BlockSpec((1, R, 1), lambda i, *_: (i, 0, 0)),
                pl.BlockSpec((D, D), lambda i, *_: (0, 0)),
                pl.BlockSpec((1, D), lambda i, *_: (0, 0)),
                pl.BlockSpec((1, D), lambda i, *_: (0, 0)),
            ],
            out_specs=pl.BlockSpec((S, D), lambda i, *_: (0, 0)),
            scratch_shapes=[
                pltpu.VMEM((S, 1), jnp.float32),
                pltpu.VMEM((S, 1), jnp.float32),
            ],
        ),
        out_shape=jax.ShapeDtypeStruct((S, D), jnp.float32),
    )(bases, lasts, x, batch3, W1, b1r, W2r)
    return out
